# baseline (speedup 1.0000x reference)
import functools

import jax
import jax.numpy as jnp
from jax import lax
from jax.experimental import pallas as pl
from jax.experimental.pallas import tpu as pltpu
from jax.experimental.pallas import tpu_sc as plsc

B = 16384
V = 1000000
D = 64
W = 128
NEG = 20

NC = 2
NS = 16
L = 16
NW = NC * NS
BW = B // NW
C = 16
NCHUNK = BW // C
H = 16


def _sc_score_body(cid_hbm, pid_hbm, nid_hbm, cw_hbm, xw_hbm,
                   pos_out, neg_out,
                   cidh_v, pidh_v, nidh_v, cpar_v, ppar_v, npar_v,
                   crow0, crow1, prow0, prow1,
                   nrow0, nrow1, psc, nsc0, nsc1, gsem0, gsem1,
                   ssem0, ssem1):
    c = lax.axis_index("c")
    s = lax.axis_index("s")
    wid = s * NC + c
    base = wid * BW
    crow = (crow0, crow1)
    prow = (prow0, prow1)
    nrow = (nrow0, nrow1)
    nsc = (nsc0, nsc1)
    gsem = (gsem0, gsem1)
    ssem = (ssem0, ssem1)

    one = jnp.full((L,), 1, jnp.int32)

    pltpu.sync_copy(cid_hbm.at[pl.ds(base, BW)], cidh_v)
    pltpu.sync_copy(pid_hbm.at[pl.ds(base, BW)], pidh_v)
    for j in range(NEG):
        pltpu.sync_copy(nid_hbm.at[j, pl.ds(base, BW)], nidh_v.at[j])

    def halve_body(i, _):
        sl = pl.ds(i * L, L)
        v = cidh_v[sl]
        cpar_v[sl] = lax.shift_left(v & one, 6)
        cidh_v[sl] = lax.shift_right_logical(v, 1)
        v = pidh_v[sl]
        ppar_v[sl] = lax.shift_left(v & one, 6)
        pidh_v[sl] = lax.shift_right_logical(v, 1)
        for j in range(NEG):
            v = nidh_v[j, sl]
            npar_v[j, sl] = lax.shift_left(v & one, 6)
            nidh_v[j, sl] = lax.shift_right_logical(v, 1)
        return 0

    lax.fori_loop(0, BW // L, halve_body, 0)

    lanes = lax.iota(jnp.int32, L)
    cols = [jnp.full((L,), d, jnp.int32) for d in range(D)]

    def chunk_copies(g, sub, make):
        cb = g * C
        f = pltpu.make_async_copy if make else pltpu.async_copy
        out = [
            f(cw_hbm.at[cidh_v.at[pl.ds(cb, C)]], crow[sub], gsem[sub]),
            f(xw_hbm.at[pidh_v.at[pl.ds(cb, C)]], prow[sub], gsem[sub]),
        ]
        for j in range(NEG):
            out.append(f(
                xw_hbm.at[nidh_v.at[j, pl.ds(cb, C)]],
                nrow[sub].at[pl.ds(j * C, C)], gsem[sub]))
        return out

    def score_copy(g, sub, make):
        f = pltpu.make_async_copy if make else pltpu.async_copy
        return f(nsc[sub],
                 neg_out.at[pl.ds(base * NEG + g * (C * NEG), C * NEG)],
                 ssem[sub])

    def compute_chunk(g, sub):
        cb = g * C
        coff = cpar_v[pl.ds(cb, L)]
        poff = ppar_v[pl.ds(cb, L)]
        ap = [jnp.zeros((L,), jnp.float32) for _ in range(4)]
        for db in range(D // H):
            ct = [plsc.load_gather(crow[sub], [lanes, coff + cols[db * H + k]])
                  for k in range(H)]
            for k in range(H):
                x = plsc.load_gather(prow[sub],
                                     [lanes, poff + cols[db * H + k]])
                ap[k % 4] = ap[k % 4] + ct[k] * x

            def jbody(j, _, db=db, ct=ct, sub=sub, cb=cb):
                noff = plsc.load_gather(npar_v,
                                        [jnp.full((L,), j, jnp.int32),
                                         lanes + cb])
                nr = lanes + j * C
                t = [jnp.zeros((L,), jnp.float32) for _ in range(4)]
                for k in range(H):
                    x = plsc.load_gather(nrow[sub],
                                         [nr, noff + cols[db * H + k]])
                    t[k % 4] = t[k % 4] + ct[k] * x
                tt = (t[0] + t[1]) + (t[2] + t[3])
                off = j * C
                if db == 0:
                    nsc[sub][pl.ds(off, L)] = tt
                else:
                    nsc[sub][pl.ds(off, L)] = nsc[sub][pl.ds(off, L)] + tt
                return 0

            lax.fori_loop(0, NEG, jbody, 0)
        psc[pl.ds(cb, L)] = (ap[0] + ap[1]) + (ap[2] + ap[3])

    chunk_copies(0, 0, make=False)
    chunk_copies(1, 1, make=False)

    def pair_body(p, carry):
        for sub in range(2):
            g = p * 2 + sub
            for cp in chunk_copies(g, sub, make=True):
                cp.wait()

            @pl.when(g >= 2)
            def _():
                score_copy(g - 2, sub, make=True).wait()

            compute_chunk(g, sub)
            score_copy(g, sub, make=False)

            @pl.when(g + 2 < NCHUNK)
            def _():
                chunk_copies(g + 2, sub, make=False)
        return carry

    lax.fori_loop(0, NCHUNK // 2, pair_body, 0)

    score_copy(NCHUNK - 2, 0, make=True).wait()
    score_copy(NCHUNK - 1, 1, make=True).wait()
    pltpu.sync_copy(psc, pos_out.at[pl.ds(base, BW)])


def _sc_score(cid, pid, nid_t, cw2, xw2):
    mesh = plsc.VectorSubcoreMesh(core_axis_name="c", subcore_axis_name="s")
    f = functools.partial(
        pl.kernel,
        mesh=mesh,
        compiler_params=pltpu.CompilerParams(
            needs_layout_passes=False, use_tc_tiling_on_sc=True),
        out_type=[
            jax.ShapeDtypeStruct((B,), jnp.float32),
            jax.ShapeDtypeStruct((B * NEG,), jnp.float32),
        ],
        scratch_types=[
            pltpu.VMEM((BW,), jnp.int32),
            pltpu.VMEM((BW,), jnp.int32),
            pltpu.VMEM((NEG, BW), jnp.int32),
            pltpu.VMEM((BW,), jnp.int32),
            pltpu.VMEM((BW,), jnp.int32),
            pltpu.VMEM((NEG, BW), jnp.int32),
            pltpu.VMEM((C, W), jnp.float32),
            pltpu.VMEM((C, W), jnp.float32),
            pltpu.VMEM((C, W), jnp.float32),
            pltpu.VMEM((C, W), jnp.float32),
            pltpu.VMEM((C * NEG, W), jnp.float32),
            pltpu.VMEM((C * NEG, W), jnp.float32),
            pltpu.VMEM((BW,), jnp.float32),
            pltpu.VMEM((C * NEG,), jnp.float32),
            pltpu.VMEM((C * NEG,), jnp.float32),
            pltpu.SemaphoreType.DMA,
            pltpu.SemaphoreType.DMA,
            pltpu.SemaphoreType.DMA,
            pltpu.SemaphoreType.DMA,
        ],
    )(_sc_score_body)
    return f(cid, pid, nid_t, cw2, xw2)


def _loss_body(pos_ref, neg_ref, out_ref):
    p = pos_ref[...]
    n = neg_ref[...]
    lsp = jnp.minimum(p, 0.0) - jnp.log1p(jnp.exp(-jnp.abs(p)))
    lsn = jnp.minimum(-n, 0.0) - jnp.log1p(jnp.exp(-jnp.abs(n)))
    out_ref[0, 0] = -(jnp.sum(lsp) + jnp.sum(lsn)) / B


def _loss(pos2d, neg2d):
    return pl.pallas_call(
        _loss_body,
        out_shape=jax.ShapeDtypeStruct((1, 1), jnp.float32),
        in_specs=[
            pl.BlockSpec(memory_space=pltpu.VMEM),
            pl.BlockSpec(memory_space=pltpu.VMEM),
        ],
        out_specs=pl.BlockSpec(memory_space=pltpu.SMEM),
    )(pos2d, neg2d)


def kernel(center_id, context_ids, negative_ids, center_w, context_w):
    cid = center_id.astype(jnp.int32)
    pid = context_ids.astype(jnp.int32)
    nid_t = negative_ids.astype(jnp.int32).T
    cw2 = center_w.reshape(V // 2, W)
    xw2 = context_w.reshape(V // 2, W)
    pos_sc, neg_sc = _sc_score(cid, pid, nid_t, cw2, xw2)
    out = _loss(pos_sc.reshape(B // 128, 128), neg_sc.reshape(B * NEG // 128, 128))
    return out[0, 0]


# speedup vs baseline: 1.0301x; 1.0301x over previous
import functools

import jax
import jax.numpy as jnp
from jax import lax
from jax.experimental import pallas as pl
from jax.experimental.pallas import tpu as pltpu
from jax.experimental.pallas import tpu_sc as plsc

B = 16384
V = 1000000
D = 64
NEG = 20

NC = 2
NS = 16
L = 16
NW = NC * NS
BW = B // NW
C = 32
NCHUNK = BW // C
GC = C // L
H = 32


def _sc_score_body(cid_hbm, pid_hbm, nid_hbm, cw_hbm, xw_hbm,
                   pos_out, neg_out,
                   cid_v, pid_v, nid_v, crow0, crow1, prow0, prow1,
                   nrow0, nrow1, psc, nsc, gsem0, gsem1):
    c = lax.axis_index("c")
    s = lax.axis_index("s")
    wid = s * NC + c
    base = wid * BW
    crow = (crow0, crow1)
    prow = (prow0, prow1)
    nrow = (nrow0, nrow1)
    gsem = (gsem0, gsem1)

    pltpu.sync_copy(cid_hbm.at[pl.ds(base, BW)], cid_v)
    pltpu.sync_copy(pid_hbm.at[pl.ds(base, BW)], pid_v)
    for j in range(NEG):
        pltpu.sync_copy(nid_hbm.at[j, pl.ds(base, BW)], nid_v.at[j])

    lanes = lax.iota(jnp.int32, L)
    cols = [jnp.full((L,), d, jnp.int32) for d in range(D)]

    def chunk_copies(g, sub, make):
        cb = g * C
        f = pltpu.make_async_copy if make else pltpu.async_copy
        out = [
            f(cw_hbm.at[cid_v.at[pl.ds(cb, C)]], crow[sub], gsem[sub]),
            f(xw_hbm.at[pid_v.at[pl.ds(cb, C)]], prow[sub], gsem[sub]),
        ]
        for j in range(NEG):
            out.append(f(
                xw_hbm.at[nid_v.at[j, pl.ds(cb, C)]],
                nrow[sub].at[pl.ds(j * C, C)], gsem[sub]))
        return out

    def compute_chunk(g, sub):
        for grp in range(GC):
            crows = lanes + grp * L
            ap = [jnp.zeros((L,), jnp.float32) for _ in range(4)]
            for db in range(D // H):
                ct = [plsc.load_gather(crow[sub], [crows, cols[db * H + k]])
                      for k in range(H)]
                for k in range(H):
                    x = plsc.load_gather(prow[sub], [crows, cols[db * H + k]])
                    ap[k % 4] = ap[k % 4] + ct[k] * x

                def jbody(jj, _, db=db, ct=ct, crows=crows, grp=grp, g=g,
                          sub=sub):
                    for j in (jj * 2, jj * 2 + 1):
                        nr = crows + j * C
                        t = [jnp.zeros((L,), jnp.float32) for _ in range(4)]
                        for k in range(H):
                            x = plsc.load_gather(nrow[sub],
                                                 [nr, cols[db * H + k]])
                            t[k % 4] = t[k % 4] + ct[k] * x
                        tt = (t[0] + t[1]) + (t[2] + t[3])
                        off = g * (C * NEG) + j * C + grp * L
                        if db == 0:
                            nsc[pl.ds(off, L)] = tt
                        else:
                            nsc[pl.ds(off, L)] = nsc[pl.ds(off, L)] + tt
                    return 0

                lax.fori_loop(0, NEG // 2, jbody, 0)
            psc[pl.ds(g * C + grp * L, L)] = (ap[0] + ap[1]) + (ap[2] + ap[3])

    chunk_copies(0, 0, make=False)
    chunk_copies(1, 1, make=False)

    def pair_body(p, carry):
        for sub in range(2):
            g = p * 2 + sub
            for cp in chunk_copies(g, sub, make=True):
                cp.wait()
            compute_chunk(g, sub)

            @pl.when(g + 2 < NCHUNK)
            def _():
                chunk_copies(g + 2, sub, make=False)
        return carry

    lax.fori_loop(0, NCHUNK // 2, pair_body, 0)

    pltpu.sync_copy(psc, pos_out.at[pl.ds(base, BW)])
    pltpu.sync_copy(nsc, neg_out.at[pl.ds(base * NEG, BW * NEG)])


def _sc_score(cid, pid, nid_t, cw, xw):
    mesh = plsc.VectorSubcoreMesh(core_axis_name="c", subcore_axis_name="s")
    f = functools.partial(
        pl.kernel,
        mesh=mesh,
        compiler_params=pltpu.CompilerParams(
            needs_layout_passes=False, use_tc_tiling_on_sc=False),
        out_type=[
            jax.ShapeDtypeStruct((B,), jnp.float32),
            jax.ShapeDtypeStruct((B * NEG,), jnp.float32),
        ],
        scratch_types=[
            pltpu.VMEM((BW,), jnp.int32),
            pltpu.VMEM((BW,), jnp.int32),
            pltpu.VMEM((NEG, BW), jnp.int32),
            pltpu.VMEM((C, D), jnp.float32),
            pltpu.VMEM((C, D), jnp.float32),
            pltpu.VMEM((C, D), jnp.float32),
            pltpu.VMEM((C, D), jnp.float32),
            pltpu.VMEM((C * NEG, D), jnp.float32),
            pltpu.VMEM((C * NEG, D), jnp.float32),
            pltpu.VMEM((BW,), jnp.float32),
            pltpu.VMEM((BW * NEG,), jnp.float32),
            pltpu.SemaphoreType.DMA,
            pltpu.SemaphoreType.DMA,
        ],
    )(_sc_score_body)
    return f(cid, pid, nid_t, cw, xw)


def _loss_body(pos_ref, neg_ref, out_ref):
    p = pos_ref[...]
    n = neg_ref[...]
    lsp = jnp.minimum(p, 0.0) - jnp.log1p(jnp.exp(-jnp.abs(p)))
    lsn = jnp.minimum(-n, 0.0) - jnp.log1p(jnp.exp(-jnp.abs(n)))
    out_ref[0, 0] = -(jnp.sum(lsp) + jnp.sum(lsn)) / B


def _loss(pos2d, neg2d):
    return pl.pallas_call(
        _loss_body,
        out_shape=jax.ShapeDtypeStruct((1, 1), jnp.float32),
        in_specs=[
            pl.BlockSpec(memory_space=pltpu.VMEM),
            pl.BlockSpec(memory_space=pltpu.VMEM),
        ],
        out_specs=pl.BlockSpec(memory_space=pltpu.SMEM),
    )(pos2d, neg2d)


def kernel(center_id, context_ids, negative_ids, center_w, context_w):
    cid = center_id.astype(jnp.int32)
    pid = context_ids.astype(jnp.int32)
    nid_t = negative_ids.astype(jnp.int32).T
    pos_sc, neg_sc = _sc_score(cid, pid, nid_t, center_w, context_w)
    out = _loss(pos_sc.reshape(B // 128, 128), neg_sc.reshape(B * NEG // 128, 128))
    return out[0, 0]
